# baseline (speedup 1.0000x reference)
import functools

import jax
import jax.numpy as jnp
from jax import lax
from jax.experimental import pallas as pl
from jax.experimental.pallas import tpu as pltpu
from jax.experimental.pallas import tpu_sc as plsc

_EMB_DIM = 64


@functools.lru_cache(maxsize=None)
def _build(R: int, T: int, D: int):
    info = plsc.get_sparse_core_info()
    NC, NS = info.num_cores, info.num_subcores
    NW = NC * NS
    assert R % (128 * NW) == 0 and D % 8 == 0
    RW = R // NW
    DT = D // 8

    mesh = plsc.VectorSubcoreMesh(core_axis_name="c", subcore_axis_name="s")

    @functools.partial(
        pl.kernel,
        out_type=jax.ShapeDtypeStruct((T, DT, R // 128, 8, 128), jnp.float32),
        mesh=mesh,
        scratch_types=[
            pltpu.VMEM((RW, T), jnp.int32),
            pltpu.VMEM((T, RW), jnp.int32),
            pltpu.VMEM((RW, D), jnp.float32),
            pltpu.VMEM((RW, D), jnp.float32),
            pltpu.VMEM((DT, 1, 8, 128), jnp.float32),
            pltpu.VMEM((DT, 1, 8, 128), jnp.float32),
            pltpu.SemaphoreType.DMA,
            pltpu.SemaphoreType.DMA,
            pltpu.SemaphoreType.DMA,
            pltpu.SemaphoreType.DMA,
        ],
        compiler_params=pltpu.CompilerParams(
            use_tc_tiling_on_sc=False, needs_layout_passes=False
        ),
    )
    def emb(table_hbm, idx_hbm, out_hbm, idxblk, idx_t, pan0, pan1,
            tp0, tp1, g0, g1, w0, w1):
        wid = lax.axis_index("s") * NC + lax.axis_index("c")
        rbase = wid * RW
        lane = jnp.arange(16, dtype=jnp.int32)

        pltpu.sync_copy(idx_hbm.at[pl.ds(rbase, RW)], idxblk)

        def idx_tr_body(t, _):
            for bb in range(RW // 16):
                v = plsc.load_gather(
                    idxblk,
                    [bb * 16 + lane, jnp.full((16,), 0, jnp.int32) + t],
                )
                idx_t[t, pl.ds(bb * 16, 16)] = v
            return _

        lax.fori_loop(0, T, idx_tr_body, 0)

        def issue_gather(t, pan, gsem):
            return pltpu.async_copy(table_hbm.at[idx_t.at[t]], pan, gsem)

        def transpose_panel(pan, tp):
            def tr_body(dblk, _):
                for dsub in range(8):
                    d16 = jnp.full((16,), 0, jnp.int32) + (dblk * 8 + dsub)
                    for bb in range(RW // 16):
                        v = plsc.load_gather(pan, [bb * 16 + lane, d16])
                        tp[dblk, 0, dsub, pl.ds(bb * 16, 16)] = v
                return _

            lax.fori_loop(0, DT, tr_body, 0)

        def issue_write(t, tp, wsem):
            return pltpu.async_copy(
                tp, out_hbm.at[t, pl.ds(0, DT), pl.ds(wid, 1)], wsem
            )

        def drain_write(tp, wsem):
            pltpu.make_async_copy(
                tp, out_hbm.at[0, pl.ds(0, DT), pl.ds(wid, 1)], wsem
            ).wait()

        issue_gather(0, pan0, g0)

        def body(i, _):
            t0 = 2 * i
            t1 = t0 + 1
            pltpu.make_async_copy(table_hbm.at[idx_t.at[t0]], pan0, g0).wait()
            issue_gather(t1, pan1, g1)

            @pl.when(i > 0)
            def _w0():
                drain_write(tp0, w0)

            transpose_panel(pan0, tp0)
            issue_write(t0, tp0, w0)

            pltpu.make_async_copy(table_hbm.at[idx_t.at[t1]], pan1, g1).wait()

            @pl.when(i + 1 < T // 2)
            def _g0():
                issue_gather(t0 + 2, pan0, g0)

            @pl.when(i > 0)
            def _w1():
                drain_write(tp1, w1)

            transpose_panel(pan1, tp1)
            issue_write(t1, tp1, w1)
            return _

        lax.fori_loop(0, T // 2, body, 0)
        drain_write(tp0, w0)
        drain_write(tp1, w1)

    return emb


def kernel(x, emb_weight):
    R, T = x.shape
    emb = _build(R, T, _EMB_DIM)
    out5 = emb(emb_weight, x.astype(jnp.int32))
    return out5.transpose(2, 4, 0, 1, 3).reshape(R, T, _EMB_DIM)


# speedup vs baseline: 1.0047x; 1.0047x over previous
import functools

import jax
import jax.numpy as jnp
from jax import lax
from jax.experimental import pallas as pl
from jax.experimental.pallas import tpu as pltpu
from jax.experimental.pallas import tpu_sc as plsc

_EMB_DIM = 64


@functools.lru_cache(maxsize=None)
def _build(R: int, T: int, D: int):
    info = plsc.get_sparse_core_info()
    NC, NS = info.num_cores, info.num_subcores
    NW = NC * NS
    assert R % (128 * NW) == 0 and D % 8 == 0
    RW = R // NW
    DT = D // 8

    mesh = plsc.VectorSubcoreMesh(core_axis_name="c", subcore_axis_name="s")

    @functools.partial(
        pl.kernel,
        out_type=jax.ShapeDtypeStruct((T, DT, R // 128, 8, 128), jnp.float32),
        mesh=mesh,
        scratch_types=[
            pltpu.VMEM((RW, T), jnp.int32),
            pltpu.VMEM((T, RW), jnp.int32),
            pltpu.VMEM((RW, D), jnp.float32),
            pltpu.VMEM((RW, D), jnp.float32),
            pltpu.VMEM((DT, 1, 8, 128), jnp.float32),
            pltpu.VMEM((DT, 1, 8, 128), jnp.float32),
            pltpu.SemaphoreType.DMA,
            pltpu.SemaphoreType.DMA,
            pltpu.SemaphoreType.DMA,
            pltpu.SemaphoreType.DMA,
        ],
        compiler_params=pltpu.CompilerParams(
            use_tc_tiling_on_sc=False,
            needs_layout_passes=False,
            disable_bounds_checks=True,
        ),
    )
    def emb(table_hbm, idx_hbm, out_hbm, idxblk, idx_t, pan0, pan1,
            tp0, tp1, g0, g1, w0, w1):
        wid = lax.axis_index("s") * NC + lax.axis_index("c")
        rbase = wid * RW
        lane = jnp.arange(16, dtype=jnp.int32)

        pltpu.sync_copy(idx_hbm.at[pl.ds(rbase, RW)], idxblk)

        def idx_tr_body(t, _):
            for bb in range(RW // 16):
                v = plsc.load_gather(
                    idxblk,
                    [bb * 16 + lane, jnp.full((16,), 0, jnp.int32) + t],
                )
                idx_t[t, pl.ds(bb * 16, 16)] = v
            return _

        lax.fori_loop(0, T, idx_tr_body, 0)

        def issue_gather(t, pan, gsem):
            return pltpu.async_copy(table_hbm.at[idx_t.at[t]], pan, gsem)

        def transpose_panel(pan, tp):
            def tr_body(dblk, _):
                for dsub in range(8):
                    d16 = jnp.full((16,), 0, jnp.int32) + (dblk * 8 + dsub)
                    for bb in range(RW // 16):
                        v = plsc.load_gather(pan, [bb * 16 + lane, d16])
                        tp[dblk, 0, dsub, pl.ds(bb * 16, 16)] = v
                return _

            lax.fori_loop(0, DT, tr_body, 0)

        def issue_write(t, tp, wsem):
            return pltpu.async_copy(
                tp, out_hbm.at[t, pl.ds(0, DT), pl.ds(wid, 1)], wsem
            )

        def drain_write(tp, wsem):
            pltpu.make_async_copy(
                tp, out_hbm.at[0, pl.ds(0, DT), pl.ds(wid, 1)], wsem
            ).wait()

        issue_gather(0, pan0, g0)

        def body(i, _):
            t0 = 2 * i
            t1 = t0 + 1
            pltpu.make_async_copy(table_hbm.at[idx_t.at[t0]], pan0, g0).wait()
            issue_gather(t1, pan1, g1)

            @pl.when(i > 0)
            def _w0():
                drain_write(tp0, w0)

            transpose_panel(pan0, tp0)
            issue_write(t0, tp0, w0)

            pltpu.make_async_copy(table_hbm.at[idx_t.at[t1]], pan1, g1).wait()

            @pl.when(i + 1 < T // 2)
            def _g0():
                issue_gather(t0 + 2, pan0, g0)

            @pl.when(i > 0)
            def _w1():
                drain_write(tp1, w1)

            transpose_panel(pan1, tp1)
            issue_write(t1, tp1, w1)
            return _

        lax.fori_loop(0, T // 2, body, 0)
        drain_write(tp0, w0)
        drain_write(tp1, w1)

    return emb


def kernel(x, emb_weight):
    R, T = x.shape
    emb = _build(R, T, _EMB_DIM)
    out5 = emb(emb_weight, x.astype(jnp.int32))
    return out5.transpose(2, 4, 0, 1, 3).reshape(R, T, _EMB_DIM)


# speedup vs baseline: 1.3548x; 1.3484x over previous
import functools

import jax
import jax.numpy as jnp
from jax import lax
from jax.experimental import pallas as pl
from jax.experimental.pallas import tpu as pltpu
from jax.experimental.pallas import tpu_sc as plsc

_EMB_DIM = 64


@functools.lru_cache(maxsize=None)
def _build(B: int, D: int):
    info = plsc.get_sparse_core_info()
    NC, NS = info.num_cores, info.num_subcores
    NW = NC * NS
    assert B % NW == 0
    b_per_w = B // NW
    C = 512
    NCH = b_per_w // C
    assert NCH * C == b_per_w

    mesh = plsc.VectorSubcoreMesh(core_axis_name="c", subcore_axis_name="s")

    @functools.partial(
        pl.kernel,
        out_type=jax.ShapeDtypeStruct((B, D), jnp.float32),
        mesh=mesh,
        scratch_types=[
            pltpu.VMEM((b_per_w,), jnp.int32),
            pltpu.VMEM((C, D), jnp.float32),
            pltpu.VMEM((C, D), jnp.float32),
            pltpu.SemaphoreType.DMA,
            pltpu.SemaphoreType.DMA,
            pltpu.SemaphoreType.DMA,
            pltpu.SemaphoreType.DMA,
        ],
        compiler_params=pltpu.CompilerParams(
            use_tc_tiling_on_sc=False, disable_bounds_checks=True
        ),
    )
    def emb(table_hbm, idx_hbm, out_hbm, idx_v, rows0, rows1, g0, g1, w0, w1):
        wid = lax.axis_index("s") * NC + lax.axis_index("c")
        base = wid * b_per_w
        bufs, gsems, wsems = [rows0, rows1], [g0, g1], [w0, w1]
        pltpu.sync_copy(idx_hbm.at[pl.ds(base, b_per_w)], idx_v)

        def issue_gather(i):
            b = i % 2
            return pltpu.async_copy(
                table_hbm.at[idx_v.at[pl.ds(i * C, C)]], bufs[b], gsems[b]
            )

        ghandle = issue_gather(0)
        whandles = [None] * NCH
        for i in range(NCH):
            b = i % 2
            ghandle.wait()
            whandles[i] = pltpu.async_copy(
                bufs[b], out_hbm.at[pl.ds(base + i * C, C)], wsems[b]
            )
            if i + 1 < NCH:
                if i >= 1:
                    whandles[i - 1].wait()
                ghandle = issue_gather(i + 1)
        whandles[NCH - 2].wait()
        whandles[NCH - 1].wait()

    return emb


def kernel(x, emb_weight):
    B = x.shape[0] * x.shape[1]
    emb = _build(B, _EMB_DIM)
    flat_idx = x.reshape(-1).astype(jnp.int32)
    out = emb(emb_weight, flat_idx)
    return out.reshape(x.shape + (_EMB_DIM,))
